# Initial kernel scaffold; baseline (speedup 1.0000x reference)
import jax
import jax.numpy as jnp
from jax.experimental import pallas as pl


def kernel(pt_fea, xy_ind, W1, W2, W3, W4, Wc, g0, b0, g1, b1, g2, b2, g3, b3):
    raise NotImplementedError("write your pallas kernel here")


# speedup vs baseline: 1.5800x; 1.5800x over previous
import functools

import numpy as np
import jax
import jax.numpy as jnp
from jax import lax
from jax.experimental import pallas as pl
from jax.experimental.pallas import tpu as pltpu

_N = 131072
_G0, _G1 = 480, 360
_NKEYS = _G1 * _G1
_MAXPT = 64
_EPS = 1e-5
_COMPRE = 256

_perm_cache = {}


def _shuffle_perm(n):
    if n not in _perm_cache:
        with jax.ensure_compile_time_eval():
            p = jax.random.permutation(jax.random.key(42), n)
        _perm_cache[n] = np.asarray(p)
    return _perm_cache[n]


def _layer_body(x_ref, w_ref, a_ref, c_ref, m_ref, z_ref, s_ref, *, relu_in,
                want_stats):
    i = pl.program_id(0)
    x = x_ref[...]
    y = x * a_ref[...] + c_ref[...]
    if relu_in:
        y = jnp.maximum(y, 0.0)
    z = lax.dot_general(y, w_ref[...], (((1,), (1,)), ((), ())),
                        preferred_element_type=jnp.float32)
    z_ref[...] = z

    @pl.when(i == 0)
    def _():
        s_ref[...] = jnp.zeros_like(s_ref)

    if want_stats:
        m = m_ref[...]
        s1 = lax.dot_general(m, z, (((0,), (0,)), ((), ())),
                             preferred_element_type=jnp.float32)
        s2 = lax.dot_general(m, z * z, (((0,), (0,)), ((), ())),
                             preferred_element_type=jnp.float32)
        s_ref[0:1, :] = s_ref[0:1, :] + s1
        s_ref[1:2, :] = s_ref[1:2, :] + s2


def _mlp_layer(x, w, a, c, mask2d, relu_in, want_stats, rows=2048):
    n, din = x.shape
    dout = w.shape[0]
    grid = n // rows
    body = functools.partial(_layer_body, relu_in=relu_in,
                             want_stats=want_stats)
    z, s = pl.pallas_call(
        body,
        grid=(grid,),
        in_specs=[
            pl.BlockSpec((rows, din), lambda i: (i, 0)),
            pl.BlockSpec((dout, din), lambda i: (0, 0)),
            pl.BlockSpec((1, din), lambda i: (0, 0)),
            pl.BlockSpec((1, din), lambda i: (0, 0)),
            pl.BlockSpec((rows, 1), lambda i: (i, 0)),
        ],
        out_specs=[
            pl.BlockSpec((rows, dout), lambda i: (i, 0)),
            pl.BlockSpec((8, dout), lambda i: (0, 0)),
        ],
        out_shape=[
            jax.ShapeDtypeStruct((n, dout), jnp.float32),
            jax.ShapeDtypeStruct((8, dout), jnp.float32),
        ],
    )(x, w, a.reshape(1, din), c.reshape(1, din), mask2d)
    return z, s


def _affine_from_stats(s, g, b, cnt):
    m = s[0] / cnt
    v = s[1] / cnt - m * m
    a = g * lax.rsqrt(v + _EPS)
    return a, b - m * a


_CROWS = 1280


def _compress_body(tab_ref, wc_ref, out_ref):
    t = tab_ref[...]
    y = lax.dot_general(wc_ref[...], t, (((1,), (1,)), ((), ())),
                        preferred_element_type=jnp.float32)
    out_ref[...] = jnp.maximum(y, 0.0)


def _compress(tab, wc):
    total = _G0 * _G1
    grid = total // _CROWS
    return pl.pallas_call(
        _compress_body,
        grid=(grid,),
        in_specs=[
            pl.BlockSpec((_CROWS, 512), lambda i: (i, 0)),
            pl.BlockSpec((_COMPRE, 512), lambda i: (0, 0)),
        ],
        out_specs=pl.BlockSpec((_COMPRE, _CROWS), lambda i: (0, i)),
        out_shape=jax.ShapeDtypeStruct((_COMPRE, total), jnp.float32),
    )(tab, wc)


def kernel(pt_fea, xy_ind, W1, W2, W3, W4, Wc, g0, b0, g1, b1, g2, b2, g3, b3):
    n = pt_fea.shape[0]
    perm = _shuffle_perm(n)

    keys = xy_ind[:, 0] * _G1 + xy_ind[:, 1]

    skeys = keys[perm]
    o2 = jnp.argsort(skeys, stable=True)
    sorted_keys = skeys[o2]
    ridx = jnp.arange(n, dtype=jnp.int32)
    prev = jnp.concatenate([sorted_keys[:1] - 1, sorted_keys[:-1]])
    is_start = sorted_keys != prev
    seg_start = lax.cummax(jnp.where(is_start, ridx, 0))
    rank_sorted = ridx - seg_start
    orig_sorted = jnp.asarray(perm)[o2]
    keep_sorted = rank_sorted < _MAXPT
    mask = jnp.zeros((n,), jnp.float32).at[orig_sorted].set(
        keep_sorted.astype(jnp.float32))
    mask2d = mask.reshape(n, 1)
    cnt = jnp.sum(mask)

    s0_1 = jnp.sum(pt_fea * mask2d, axis=0)
    s0_2 = jnp.sum(pt_fea * pt_fea * mask2d, axis=0)
    m0 = s0_1 / cnt
    v0 = s0_2 / cnt - m0 * m0
    a0 = g0 * lax.rsqrt(v0 + _EPS)
    c0 = b0 - m0 * a0

    z1, s1 = _mlp_layer(pt_fea, W1, a0, c0, mask2d, False, True)
    a1, c1 = _affine_from_stats(s1, g1, b1, cnt)
    z2, s2 = _mlp_layer(z1, W2, a1, c1, mask2d, True, True)
    a2, c2 = _affine_from_stats(s2, g2, b2, cnt)
    z3, s3 = _mlp_layer(z2, W3, a2, c2, mask2d, True, True)
    a3, c3 = _affine_from_stats(s3, g3, b3, cnt)
    z4, _ = _mlp_layer(z3, W4, a3, c3, mask2d, True, False)

    neg = jnp.float32(-jnp.inf)
    total = _G0 * _G1
    z4k = jnp.where(mask2d > 0, z4, neg)
    tab = jnp.full((total, 512), neg, jnp.float32).at[keys].max(z4k)
    occ = jnp.zeros((total,), jnp.int32).at[keys].add(1) > 0
    tab = jnp.where(occ[:, None], tab, 0.0)

    out = _compress(tab, Wc)
    return out.reshape(1, _COMPRE, _G0, _G1)
